# baseline (speedup 1.0000x reference)
import functools

import jax
import jax.numpy as jnp
from jax import lax
from jax.experimental import pallas as pl
from jax.experimental.pallas import tpu as pltpu
from jax.experimental.pallas import tpu_sc as plsc

B, L, D, V = 1024, 20, 128, 100000
ALPHA = 0.2
SUB = 8
R = SUB * L
STEPS = 16
SPS = B // STEPS
NSUB = SPS // SUB
RPS = SPS * L
BL = B * L

_NC = 2
_NS = 16
_NW = _NC * _NS
_B_PER_W = BL // _NW
_CHUNK = 128
_NCHUNK = _B_PER_W // _CHUNK


def _sc_gather(idx, table):
    mesh = plsc.VectorSubcoreMesh(core_axis_name="c", subcore_axis_name="s")

    @functools.partial(
        pl.kernel,
        mesh=mesh,
        out_type=jax.ShapeDtypeStruct((BL, D), jnp.float32),
        scratch_types=[
            pltpu.VMEM((_B_PER_W,), jnp.int32),
            pltpu.VMEM((_B_PER_W, D), jnp.float32),
            pltpu.SemaphoreType.DMA,
        ],
    )
    def gather_kernel(idx_hbm, table_hbm, out_hbm, idx_v, rows_v, sem):
        wid = lax.axis_index("s") * _NC + lax.axis_index("c")
        base = wid * _B_PER_W
        pltpu.sync_copy(idx_hbm.at[pl.ds(base, _B_PER_W)], idx_v)
        copies = []
        for j in range(_NCHUNK):
            copies.append(
                pltpu.async_copy(
                    table_hbm.at[idx_v.at[pl.ds(j * _CHUNK, _CHUNK)]],
                    rows_v.at[pl.ds(j * _CHUNK, _CHUNK)],
                    sem,
                )
            )
        for cp in copies:
            cp.wait()
        pltpu.sync_copy(rows_v, out_hbm.at[pl.ds(base, _B_PER_W)])

    return gather_kernel(idx, table)


def _tc_body(h_ref, adj_ref, a_ref, tile_ref, cross_ref, o_ref, hnb_scr, code_scr):
    a = a_ref[...].astype(jnp.bfloat16)
    tile = tile_ref[...]
    cross = cross_ref[...]

    h = h_ref[...]
    ss = jnp.sum(h * h, axis=1, keepdims=True)
    hn = h / jnp.maximum(jnp.sqrt(ss), 1e-12)
    hnb_scr[...] = hn.astype(jnp.bfloat16)
    adjf = adj_ref[...].astype(jnp.float32)
    code_scr[...] = lax.dot_general(adjf, tile[0:L, :], (((1,), (0,)), ((), ())),
                                    preferred_element_type=jnp.float32)

    def sub_block(s, _):
        base = pl.multiple_of(s * R, R)
        sbase = pl.multiple_of(s * SUB, SUB)
        hnb = hnb_scr[pl.ds(base, R), :]
        codef = code_scr[pl.ds(base, R), :] + cross
        pre = jnp.where(codef >= 99.5, -600.0, -300.0)
        for k in range(4):
            hk = hnb * a[k : k + 1, :]
            pk = lax.dot_general(hk, hnb, (((1,), (1,)), ((), ())),
                                 preferred_element_type=jnp.float32)
            pre = jnp.where(codef == (k + 1), pk, pre)
        alph = jnp.where(pre >= 0, pre, ALPHA * pre)
        ex = jnp.exp(alph)
        den = jnp.sum(ex, axis=1, keepdims=True)
        p = (ex / den).astype(jnp.bfloat16)
        res = lax.dot_general(
            p, hnb, (((1,), (0,)), ((), ())), preferred_element_type=jnp.float32)
        o_ref[pl.ds(sbase, SUB), :, :] = res.reshape(SUB, L, D)
        return 0

    lax.fori_loop(0, NSUB, sub_block, 0, unroll=True)


def _tc_attention(h_raw, adj2, a_mat, tile_c, cross_c):
    return pl.pallas_call(
        _tc_body,
        grid=(STEPS,),
        in_specs=[
            pl.BlockSpec((RPS, D), lambda i: (i, 0)),
            pl.BlockSpec((RPS, L), lambda i: (i, 0)),
            pl.BlockSpec((8, D), lambda i: (0, 0)),
            pl.BlockSpec((24, R), lambda i: (0, 0)),
            pl.BlockSpec((R, R), lambda i: (0, 0)),
        ],
        out_specs=pl.BlockSpec((SPS, L, D), lambda i: (i, 0, 0)),
        out_shape=jax.ShapeDtypeStruct((B, L, D), jnp.float32),
        scratch_shapes=[
            pltpu.VMEM((RPS, D), jnp.bfloat16),
            pltpu.VMEM((RPS, R), jnp.float32),
        ],
    )(h_raw, adj2, a_mat, tile_c, cross_c)


def _constants():
    j = jnp.arange(24)[:, None]
    c = jnp.arange(R)[None, :]
    tile_c = (c % L == j).astype(jnp.float32)
    rs = jnp.arange(R)[:, None] // L
    cs = jnp.arange(R)[None, :] // L
    cross_c = jnp.where(rs == cs, 0.0, 100.0).astype(jnp.float32)
    return tile_c, cross_c


def kernel(inputs, adj, mask_item, item, embedding, a_0, a_1, a_2, a_3):
    idx = inputs.reshape(BL).astype(jnp.int32)
    h_raw = _sc_gather(idx, embedding)
    adj2 = adj.reshape(BL, L)
    a_mat = jnp.concatenate(
        [a_0.T, a_1.T, a_2.T, a_3.T, jnp.zeros((4, D), jnp.float32)], axis=0)
    tile_c, cross_c = _constants()
    return _tc_attention(h_raw, adj2, a_mat, tile_c, cross_c)


# speedup vs baseline: 1.1305x; 1.1305x over previous
import functools

import jax
import jax.numpy as jnp
from jax import lax
from jax.experimental import pallas as pl
from jax.experimental.pallas import tpu as pltpu
from jax.experimental.pallas import tpu_sc as plsc

B, L, D, V = 1024, 20, 128, 100000
ALPHA = 0.2
SUB = 8
R = SUB * L
STEPS = 16
SPS = B // STEPS
NSUB = SPS // SUB
RPS = SPS * L
BL = B * L

_NC = 2
_NS = 16
_NW = _NC * _NS
_B_PER_W = BL // _NW
_CHUNK = 128
_NCHUNK = _B_PER_W // _CHUNK


def _sc_gather(idx, table):
    mesh = plsc.VectorSubcoreMesh(core_axis_name="c", subcore_axis_name="s")

    @functools.partial(
        pl.kernel,
        mesh=mesh,
        out_type=jax.ShapeDtypeStruct((BL, D), jnp.float32),
        scratch_types=[
            pltpu.VMEM((_B_PER_W,), jnp.int32),
            pltpu.VMEM((_B_PER_W, D), jnp.float32),
            pltpu.SemaphoreType.DMA,
        ],
    )
    def gather_kernel(idx_hbm, table_hbm, out_hbm, idx_v, rows_v, sem):
        wid = lax.axis_index("s") * _NC + lax.axis_index("c")
        base = wid * _B_PER_W
        pltpu.sync_copy(idx_hbm.at[pl.ds(base, _B_PER_W)], idx_v)
        copies = []
        for j in range(_NCHUNK):
            copies.append(
                pltpu.async_copy(
                    table_hbm.at[idx_v.at[pl.ds(j * _CHUNK, _CHUNK)]],
                    rows_v.at[pl.ds(j * _CHUNK, _CHUNK)],
                    sem,
                )
            )
        for cp in copies:
            cp.wait()
        pltpu.sync_copy(rows_v, out_hbm.at[pl.ds(base, _B_PER_W)])

    return gather_kernel(idx, table)


def _tc_body(h_ref, adj_ref, a_ref, tile_ref, cross_ref, o_ref, hnb_scr, code_scr):
    a = a_ref[...].astype(jnp.bfloat16)
    tile = tile_ref[...]
    cross = cross_ref[...]

    h = h_ref[...]
    ss = jnp.sum(h * h, axis=1, keepdims=True)
    hn = h / jnp.maximum(jnp.sqrt(ss), 1e-12)
    hnb_scr[...] = hn.astype(jnp.bfloat16)
    adjf = adj_ref[...].reshape(RPS, L).astype(jnp.float32)
    code_scr[...] = lax.dot_general(adjf, tile[0:L, :], (((1,), (0,)), ((), ())),
                                    preferred_element_type=jnp.float32)

    def sub_block(s, _):
        base = pl.multiple_of(s * R, R)
        sbase = pl.multiple_of(s * SUB, SUB)
        hnb = hnb_scr[pl.ds(base, R), :]
        codef = code_scr[pl.ds(base, R), :] + cross
        pre = jnp.where(codef >= 99.5, -600.0, -300.0)
        for k in range(4):
            hk = hnb * a[k : k + 1, :]
            pk = lax.dot_general(hk, hnb, (((1,), (1,)), ((), ())),
                                 preferred_element_type=jnp.float32)
            pre = jnp.where(codef == (k + 1), pk, pre)
        alph = jnp.where(pre >= 0, pre, ALPHA * pre)
        ex = jnp.exp(alph)
        den = jnp.sum(ex, axis=1, keepdims=True)
        p = (ex / den).astype(jnp.bfloat16)
        res = lax.dot_general(
            p, hnb, (((1,), (0,)), ((), ())), preferred_element_type=jnp.float32)
        o_ref[pl.ds(sbase, SUB), :, :] = res.reshape(SUB, L, D)
        return 0

    lax.fori_loop(0, NSUB, sub_block, 0, unroll=True)


def _tc_attention(h_raw, adj2, a_mat, tile_c, cross_c):
    return pl.pallas_call(
        _tc_body,
        grid=(STEPS,),
        in_specs=[
            pl.BlockSpec((RPS, D), lambda i: (i, 0)),
            pl.BlockSpec((SPS, L, L), lambda i: (i, 0, 0)),
            pl.BlockSpec((8, D), lambda i: (0, 0)),
            pl.BlockSpec((24, R), lambda i: (0, 0)),
            pl.BlockSpec((R, R), lambda i: (0, 0)),
        ],
        out_specs=pl.BlockSpec((SPS, L, D), lambda i: (i, 0, 0)),
        out_shape=jax.ShapeDtypeStruct((B, L, D), jnp.float32),
        scratch_shapes=[
            pltpu.VMEM((RPS, D), jnp.bfloat16),
            pltpu.VMEM((RPS, R), jnp.float32),
        ],
    )(h_raw, adj2, a_mat, tile_c, cross_c)


def _constants():
    j = jnp.arange(24)[:, None]
    c = jnp.arange(R)[None, :]
    tile_c = (c % L == j).astype(jnp.float32)
    rs = jnp.arange(R)[:, None] // L
    cs = jnp.arange(R)[None, :] // L
    cross_c = jnp.where(rs == cs, 0.0, 100.0).astype(jnp.float32)
    return tile_c, cross_c


def kernel(inputs, adj, mask_item, item, embedding, a_0, a_1, a_2, a_3):
    idx = inputs.reshape(BL).astype(jnp.int32)
    h_raw = _sc_gather(idx, embedding)
    a_mat = jnp.concatenate(
        [a_0.T, a_1.T, a_2.T, a_3.T, jnp.zeros((4, D), jnp.float32)], axis=0)
    tile_c, cross_c = _constants()
    return _tc_attention(h_raw, adj, a_mat, tile_c, cross_c)


# speedup vs baseline: 1.1357x; 1.0046x over previous
import functools

import jax
import jax.numpy as jnp
from jax import lax
from jax.experimental import pallas as pl
from jax.experimental.pallas import tpu as pltpu
from jax.experimental.pallas import tpu_sc as plsc

B, L, D, V = 1024, 20, 128, 100000
ALPHA = 0.2
SUB = 8
R = SUB * L
STEPS = 16
SPS = B // STEPS
NSUB = SPS // SUB
RPS = SPS * L
BL = B * L

_NC = 2
_NS = 16
_NW = _NC * _NS
_B_PER_W = BL // _NW
_CHUNK = 128
_NCHUNK = _B_PER_W // _CHUNK


def _sc_gather(idx, table):
    mesh = plsc.VectorSubcoreMesh(core_axis_name="c", subcore_axis_name="s")

    @functools.partial(
        pl.kernel,
        mesh=mesh,
        out_type=jax.ShapeDtypeStruct((BL, D), jnp.float32),
        scratch_types=[
            pltpu.VMEM((_B_PER_W,), jnp.int32),
            pltpu.VMEM((_B_PER_W, D), jnp.float32),
            pltpu.SemaphoreType.DMA,
        ],
    )
    def gather_kernel(idx_hbm, table_hbm, out_hbm, idx_v, rows_v, sem):
        wid = lax.axis_index("s") * _NC + lax.axis_index("c")
        base = wid * _B_PER_W
        pltpu.sync_copy(idx_hbm.at[pl.ds(base, _B_PER_W)], idx_v)
        copies = []
        for j in range(_NCHUNK):
            copies.append(
                pltpu.async_copy(
                    table_hbm.at[idx_v.at[pl.ds(j * _CHUNK, _CHUNK)]],
                    rows_v.at[pl.ds(j * _CHUNK, _CHUNK)],
                    sem,
                )
            )
        for cp in copies:
            cp.wait()
        pltpu.sync_copy(rows_v, out_hbm.at[pl.ds(base, _B_PER_W)])

    return gather_kernel(idx, table)


def _tc_body(h_ref, adj_ref, a_ref, tile_ref, cross_ref, perm_ref, o_ref,
             hnb_scr, code_scr):
    a = a_ref[...].astype(jnp.bfloat16)
    tile = tile_ref[...]
    cross = cross_ref[...]
    permb = perm_ref[...].astype(jnp.bfloat16)

    h = h_ref[...]
    ss = jnp.sum(h * h, axis=1, keepdims=True)
    hn = h / jnp.maximum(jnp.sqrt(ss), 1e-12)
    hnb = hn.astype(jnp.bfloat16)
    adjb = adj_ref[...].reshape(RPS, L).astype(jnp.bfloat16)
    for s in range(NSUB):
        base = s * R
        hp = lax.dot_general(permb, hnb[base : base + R, :],
                             (((1,), (0,)), ((), ())),
                             preferred_element_type=jnp.float32)
        hnb_scr[pl.ds(base, R), :] = hp.astype(jnp.bfloat16)
        ap = lax.dot_general(permb, adjb[base : base + R, :],
                             (((1,), (0,)), ((), ())),
                             preferred_element_type=jnp.float32)
        code_scr[pl.ds(base, R), :] = lax.dot_general(
            ap, tile[0:L, :], (((1,), (0,)), ((), ())),
            preferred_element_type=jnp.float32)

    def sub_block(s, _):
        base = pl.multiple_of(s * R, R)
        sbase = pl.multiple_of(s * SUB, SUB)
        hnp = hnb_scr[pl.ds(base, R), :]
        codef = code_scr[pl.ds(base, R), :] + cross
        pre = jnp.where(codef >= 99.5, -600.0, -300.0)
        for k in range(4):
            hk = hnp * a[k : k + 1, :]
            pk = lax.dot_general(hk, hnp, (((1,), (1,)), ((), ())),
                                 preferred_element_type=jnp.float32)
            pre = jnp.where(codef == (k + 1), pk, pre)
        alph = jnp.where(pre >= 0, pre, ALPHA * pre)
        ex = jnp.exp(alph)
        den = jnp.sum(ex, axis=1, keepdims=True)
        p = (ex / den).astype(jnp.bfloat16)
        res = lax.dot_general(
            p, hnp, (((1,), (0,)), ((), ())), preferred_element_type=jnp.float32)
        o_ref[:, pl.ds(sbase, SUB), :] = res.reshape(L, SUB, D)
        return 0

    lax.fori_loop(0, NSUB, sub_block, 0, unroll=True)


def _tc_attention(h_raw, adj, a_mat, tile_c, cross_c, perm_c):
    return pl.pallas_call(
        _tc_body,
        grid=(STEPS,),
        in_specs=[
            pl.BlockSpec((RPS, D), lambda i: (i, 0)),
            pl.BlockSpec((SPS, L, L), lambda i: (i, 0, 0)),
            pl.BlockSpec((8, D), lambda i: (0, 0)),
            pl.BlockSpec((24, R), lambda i: (0, 0)),
            pl.BlockSpec((R, R), lambda i: (0, 0)),
            pl.BlockSpec((R, R), lambda i: (0, 0)),
        ],
        out_specs=pl.BlockSpec((L, SPS, D), lambda i: (0, i, 0)),
        out_shape=jax.ShapeDtypeStruct((L, B, D), jnp.float32),
        scratch_shapes=[
            pltpu.VMEM((RPS, D), jnp.bfloat16),
            pltpu.VMEM((RPS, R), jnp.float32),
        ],
    )(h_raw, adj, a_mat, tile_c, cross_c, perm_c)


def _constants():
    rr = jnp.arange(R)
    sess_r = rr % SUB
    item_r = rr // SUB
    perm_c = (jnp.arange(R)[None, :] == (sess_r * L + item_r)[:, None])
    perm_c = perm_c.astype(jnp.float32)
    j = jnp.arange(24)[:, None]
    tile_c = (item_r[None, :] == j).astype(jnp.float32)
    cross_c = jnp.where(sess_r[:, None] == sess_r[None, :], 0.0, 100.0)
    cross_c = cross_c.astype(jnp.float32)
    return tile_c, cross_c, perm_c


def kernel(inputs, adj, mask_item, item, embedding, a_0, a_1, a_2, a_3):
    idx = inputs.reshape(BL).astype(jnp.int32)
    h_raw = _sc_gather(idx, embedding)
    a_mat = jnp.concatenate(
        [a_0.T, a_1.T, a_2.T, a_3.T, jnp.zeros((4, D), jnp.float32)], axis=0)
    tile_c, cross_c, perm_c = _constants()
    out = _tc_attention(h_raw, adj, a_mat, tile_c, cross_c, perm_c)
    return jnp.transpose(out, (1, 0, 2))


# speedup vs baseline: 1.2475x; 1.0985x over previous
import functools

import jax
import jax.numpy as jnp
from jax import lax
from jax.experimental import pallas as pl
from jax.experimental.pallas import tpu as pltpu
from jax.experimental.pallas import tpu_sc as plsc

B, L, D, V = 1024, 20, 128, 100000
ALPHA = 0.2
SUB = 8
R = SUB * L
STEPS = 16
SPS = B // STEPS
NSUB = SPS // SUB
RPS = SPS * L
BL = B * L

_NC = 2
_NS = 16
_NW = _NC * _NS
_B_PER_W = BL // _NW
_CHUNK = 128
_NCHUNK = _B_PER_W // _CHUNK


def _sc_gather(idx, table):
    mesh = plsc.VectorSubcoreMesh(core_axis_name="c", subcore_axis_name="s")

    @functools.partial(
        pl.kernel,
        mesh=mesh,
        out_type=jax.ShapeDtypeStruct((BL, D), jnp.float32),
        scratch_types=[
            pltpu.VMEM((_B_PER_W,), jnp.int32),
            pltpu.VMEM((_B_PER_W, D), jnp.float32),
            pltpu.SemaphoreType.DMA,
        ],
    )
    def gather_kernel(idx_hbm, table_hbm, out_hbm, idx_v, rows_v, sem):
        wid = lax.axis_index("s") * _NC + lax.axis_index("c")
        base = wid * _B_PER_W
        pltpu.sync_copy(idx_hbm.at[pl.ds(base, _B_PER_W)], idx_v)
        copies = []
        for j in range(_NCHUNK):
            copies.append(
                pltpu.async_copy(
                    table_hbm.at[idx_v.at[pl.ds(j * _CHUNK, _CHUNK)]],
                    rows_v.at[pl.ds(j * _CHUNK, _CHUNK)],
                    sem,
                )
            )
        for cp in copies:
            cp.wait()
        pltpu.sync_copy(rows_v, out_hbm.at[pl.ds(base, _B_PER_W)])

    return gather_kernel(idx, table)


def _tc_body(h_ref, adj_ref, a_ref, tile_ref, cross_ref, perm_ref, o_ref,
             hnb_scr, code_scr):
    a = a_ref[...].astype(jnp.bfloat16)
    tile = tile_ref[...]
    cross = cross_ref[...]
    permb = perm_ref[...].astype(jnp.bfloat16)

    h = h_ref[...]
    ss = jnp.sum(h * h, axis=1, keepdims=True)
    hn = h / jnp.maximum(jnp.sqrt(ss), 1e-12)
    hnb = hn.astype(jnp.bfloat16)
    hnb_scr[...] = hnb
    adjf = adj_ref[...].reshape(RPS, L).astype(jnp.float32)
    code_scr[...] = lax.dot_general(adjf, tile[0:L, :], (((1,), (0,)), ((), ())),
                                    preferred_element_type=jnp.float32)

    def sub_block(s, _):
        base = pl.multiple_of(s * R, R)
        sbase = pl.multiple_of(s * SUB, SUB)
        hnp = hnb_scr[pl.ds(base, R), :]
        codef = code_scr[pl.ds(base, R), :] + cross
        pre = jnp.where(codef >= 99.5, -600.0, -300.0)
        for k in range(4):
            hk = hnp * a[k : k + 1, :]
            pk = lax.dot_general(hk, hnp, (((1,), (1,)), ((), ())),
                                 preferred_element_type=jnp.float32)
            pre = jnp.where(codef == (k + 1), pk, pre)
        alph = jnp.where(pre >= 0, pre, ALPHA * pre)
        ex = jnp.exp(alph)
        den = jnp.sum(ex, axis=1, keepdims=True)
        p = (ex / den).astype(jnp.bfloat16)
        res = lax.dot_general(
            p, hnp, (((1,), (0,)), ((), ())), preferred_element_type=jnp.float32)
        o_ref[:, pl.ds(sbase, SUB), :] = res.reshape(SUB, L, D).transpose(1, 0, 2)
        return 0

    lax.fori_loop(0, NSUB, sub_block, 0, unroll=True)


def _tc_attention(h_raw, adj, a_mat, tile_c, cross_c, perm_c):
    return pl.pallas_call(
        _tc_body,
        grid=(STEPS,),
        in_specs=[
            pl.BlockSpec((RPS, D), lambda i: (i, 0)),
            pl.BlockSpec((SPS, L, L), lambda i: (i, 0, 0)),
            pl.BlockSpec((8, D), lambda i: (0, 0)),
            pl.BlockSpec((24, R), lambda i: (0, 0)),
            pl.BlockSpec((R, R), lambda i: (0, 0)),
            pl.BlockSpec((R, R), lambda i: (0, 0)),
        ],
        out_specs=pl.BlockSpec((L, SPS, D), lambda i: (0, i, 0)),
        out_shape=jax.ShapeDtypeStruct((L, B, D), jnp.float32),
        scratch_shapes=[
            pltpu.VMEM((RPS, D), jnp.bfloat16),
            pltpu.VMEM((RPS, R), jnp.float32),
        ],
    )(h_raw, adj, a_mat, tile_c, cross_c, perm_c)


def _constants():
    j = jnp.arange(24)[:, None]
    c = jnp.arange(R)[None, :]
    tile_c = (c % L == j).astype(jnp.float32)
    rs = jnp.arange(R)[:, None] // L
    cs = jnp.arange(R)[None, :] // L
    cross_c = jnp.where(rs == cs, 0.0, 100.0).astype(jnp.float32)
    perm_c = jnp.zeros((R, R), jnp.float32)
    return tile_c, cross_c, perm_c


def kernel(inputs, adj, mask_item, item, embedding, a_0, a_1, a_2, a_3):
    idx = inputs.reshape(BL).astype(jnp.int32)
    h_raw = _sc_gather(idx, embedding)
    a_mat = jnp.concatenate(
        [a_0.T, a_1.T, a_2.T, a_3.T, jnp.zeros((4, D), jnp.float32)], axis=0)
    tile_c, cross_c, perm_c = _constants()
    out = _tc_attention(h_raw, adj, a_mat, tile_c, cross_c, perm_c)
    return jnp.transpose(out, (1, 0, 2))


# speedup vs baseline: 1.2990x; 1.0413x over previous
import functools

import jax
import jax.numpy as jnp
from jax import lax
from jax.experimental import pallas as pl
from jax.experimental.pallas import tpu as pltpu
from jax.experimental.pallas import tpu_sc as plsc

B, L, D, V = 1024, 20, 128, 100000
ALPHA = 0.2
SUB = 8
R = SUB * L
STEPS = 8
SPS = B // STEPS
NSUB = SPS // SUB
RPS = SPS * L
BL = B * L

_NC = 2
_NS = 16
_NW = _NC * _NS
_B_PER_W = BL // _NW
_CHUNK = 128
_NCHUNK = _B_PER_W // _CHUNK


def _sc_gather(idx, table):
    mesh = plsc.VectorSubcoreMesh(core_axis_name="c", subcore_axis_name="s")

    @functools.partial(
        pl.kernel,
        mesh=mesh,
        out_type=jax.ShapeDtypeStruct((BL, D), jnp.float32),
        scratch_types=[
            pltpu.VMEM((_B_PER_W,), jnp.int32),
            pltpu.VMEM((_B_PER_W, D), jnp.float32),
            pltpu.SemaphoreType.DMA,
        ],
    )
    def gather_kernel(idx_hbm, table_hbm, out_hbm, idx_v, rows_v, sem):
        wid = lax.axis_index("s") * _NC + lax.axis_index("c")
        base = wid * _B_PER_W
        pltpu.sync_copy(idx_hbm.at[pl.ds(base, _B_PER_W)], idx_v)
        copies = []
        for j in range(_NCHUNK):
            copies.append(
                pltpu.async_copy(
                    table_hbm.at[idx_v.at[pl.ds(j * _CHUNK, _CHUNK)]],
                    rows_v.at[pl.ds(j * _CHUNK, _CHUNK)],
                    sem,
                )
            )
        for cp in copies:
            cp.wait()
        pltpu.sync_copy(rows_v, out_hbm.at[pl.ds(base, _B_PER_W)])

    return gather_kernel(idx, table)


def _tc_body(h_ref, adj_ref, a_ref, tile_ref, cross_ref, o_ref,
             hnb_scr, code_scr):
    a = a_ref[...].astype(jnp.bfloat16)
    tile = tile_ref[...]
    cross = cross_ref[...]

    h = h_ref[...]
    ss = jnp.sum(h * h, axis=1, keepdims=True)
    hn = h / jnp.maximum(jnp.sqrt(ss), 1e-12)
    hnb = hn.astype(jnp.bfloat16)
    hnb_scr[...] = hnb
    adjf = adj_ref[...].reshape(RPS, L).astype(jnp.float32)
    code_scr[...] = lax.dot_general(adjf, tile[0:L, :], (((1,), (0,)), ((), ())),
                                    preferred_element_type=jnp.float32)

    def sub_block(s, _):
        base = pl.multiple_of(s * R, R)
        sbase = pl.multiple_of(s * SUB, SUB)
        hnp = hnb_scr[pl.ds(base, R), :]
        codef = code_scr[pl.ds(base, R), :] + cross
        pre = jnp.where(codef >= 99.5, -600.0, -300.0)
        for k in range(4):
            hk = hnp * a[k : k + 1, :]
            pk = lax.dot_general(hk, hnp, (((1,), (1,)), ((), ())),
                                 preferred_element_type=jnp.float32)
            pre = jnp.where(codef == (k + 1), pk, pre)
        alph = jnp.where(pre >= 0, pre, ALPHA * pre)
        ex = jnp.exp(alph)
        den = jnp.sum(ex, axis=1, keepdims=True)
        p = (ex / den).astype(jnp.bfloat16)
        res = lax.dot_general(
            p, hnp, (((1,), (0,)), ((), ())), preferred_element_type=jnp.float32)
        o_ref[:, pl.ds(sbase, SUB), :] = res.reshape(SUB, L, D).transpose(1, 0, 2)
        return 0

    lax.fori_loop(0, NSUB, sub_block, 0, unroll=True)


def _tc_attention(h_raw, adj, a_mat, tile_c, cross_c):
    return pl.pallas_call(
        _tc_body,
        grid=(STEPS,),
        in_specs=[
            pl.BlockSpec((RPS, D), lambda i: (i, 0)),
            pl.BlockSpec((SPS, L, L), lambda i: (i, 0, 0)),
            pl.BlockSpec((8, D), lambda i: (0, 0)),
            pl.BlockSpec((24, R), lambda i: (0, 0)),
            pl.BlockSpec((R, R), lambda i: (0, 0)),
        ],
        out_specs=pl.BlockSpec((L, SPS, D), lambda i: (0, i, 0)),
        out_shape=jax.ShapeDtypeStruct((L, B, D), jnp.float32),
        scratch_shapes=[
            pltpu.VMEM((RPS, D), jnp.bfloat16),
            pltpu.VMEM((RPS, R), jnp.float32),
        ],
    )(h_raw, adj, a_mat, tile_c, cross_c)


def _constants():
    j = jnp.arange(24)[:, None]
    c = jnp.arange(R)[None, :]
    tile_c = (c % L == j).astype(jnp.float32)
    rs = jnp.arange(R)[:, None] // L
    cs = jnp.arange(R)[None, :] // L
    cross_c = jnp.where(rs == cs, 0.0, 100.0).astype(jnp.float32)
    return tile_c, cross_c


def kernel(inputs, adj, mask_item, item, embedding, a_0, a_1, a_2, a_3):
    idx = inputs.reshape(BL).astype(jnp.int32)
    h_raw = _sc_gather(idx, embedding)
    a_mat = jnp.concatenate(
        [a_0.T, a_1.T, a_2.T, a_3.T, jnp.zeros((4, D), jnp.float32)], axis=0)
    tile_c, cross_c = _constants()
    out = _tc_attention(h_raw, adj, a_mat, tile_c, cross_c)
    return jnp.transpose(out, (1, 0, 2))
